# Initial kernel scaffold; baseline (speedup 1.0000x reference)
import jax
import jax.numpy as jnp
from jax.experimental import pallas as pl


def kernel(x, edge_index, W1, b1, W2, b2, Wfc, bfc):
    raise NotImplementedError("write your pallas kernel here")


# speedup vs baseline: 4.6978x; 4.6978x over previous
import functools

import jax
import jax.numpy as jnp
from jax import lax
from jax.experimental import pallas as pl
from jax.experimental.pallas import tpu as pltpu
from jax.experimental.pallas import tpu_sc as plsc

N = 10000
E = 320000
D = 128
NC, NS = 2, 16
NW = NC * NS
CH = 128
NCHUNKS = E // CH
CPW = NCHUNKS // NW
EXTRA = NCHUNKS - CPW * NW
BR = 400
NRC = N // BR

_mesh = plsc.VectorSubcoreMesh(core_axis_name="c", subcore_axis_name="s")


def _worker_id():
    return lax.axis_index("s") * NC + lax.axis_index("c")


def _copy_rows(src_at, dst_at, base, total, piece):
    off = 0
    while off < total:
        n = min(piece, total - off)
        pltpu.sync_copy(src_at(base + off, n), dst_at(base + off, n))
        off += n


def _over_row_chunks(s, fn):
    fn(s * BR)

    @pl.when(s < NRC - NS)
    def _():
        fn((s + NS) * BR)


NPAD = 10240
PK = NPAD // 8
PKS = PK // NS


@functools.partial(
    pl.kernel,
    out_type=jax.ShapeDtypeStruct((NC, PK, 128), jnp.float32),
    mesh=_mesh,
    scratch_types=[
        pltpu.VMEM((CH,), jnp.int32),
        pltpu.VMEM((CH,), jnp.int32),
        pltpu.VMEM((CH,), jnp.int32),
        pltpu.VMEM((CH, D), jnp.float32),
        pltpu.VMEM_SHARED((PK, 128), jnp.float32),
    ],
)
def _deg_sc(dst_hbm, table_hbm, out_hbm, idx_v, idxhi, idxlo, rows, acc):
    c = lax.axis_index("c")
    s = lax.axis_index("s")
    wid = _worker_id()

    zero16 = jnp.zeros((16,), jnp.float32)

    def zrow(i, _):
        for j in range(D // 16):
            rows[i, pl.ds(j * 16, 16)] = zero16
        return _

    lax.fori_loop(0, CH, zrow, 0)
    pltpu.sync_copy(rows.at[pl.ds(0, PKS)], acc.at[pl.ds(s * PKS, PKS)])
    plsc.subcore_barrier()

    cbase = wid * CPW

    def chunk(k):
        off = k * CH
        pltpu.sync_copy(dst_hbm.at[pl.ds(off, CH)], idx_v)
        for j in range(CH // 16):
            v = idx_v[pl.ds(j * 16, 16)]
            idxhi[pl.ds(j * 16, 16)] = lax.shift_right_logical(v, 3)
            idxlo[pl.ds(j * 16, 16)] = lax.bitwise_and(v, 7)
        pltpu.sync_copy(table_hbm.at[idxlo], rows)
        pltpu.sync_copy(rows, acc.at[idxhi], add=True)

    def body(k, _):
        chunk(cbase + k)
        return _

    lax.fori_loop(0, CPW, body, 0)

    @pl.when(wid < EXTRA)
    def _():
        chunk(NW * CPW + wid)

    plsc.subcore_barrier()
    pltpu.sync_copy(acc.at[pl.ds(s * PKS, PKS)], out_hbm.at[c, pl.ds(s * PKS, PKS)])


@functools.partial(
    pl.kernel,
    out_type=jax.ShapeDtypeStruct((NC, N, D), jnp.float32),
    mesh=_mesh,
    scratch_types=[
        pltpu.VMEM((CH,), jnp.int32),
        pltpu.VMEM((CH,), jnp.int32),
        pltpu.VMEM((CH, D), jnp.float32),
        pltpu.VMEM_SHARED((N, D), jnp.float32),
    ],
)
def _agg_sc(g_hbm, src_hbm, dst_hbm, out_hbm, idx_s, idx_d, rows, acc):
    c = lax.axis_index("c")
    s = lax.axis_index("s")
    wid = _worker_id()

    zero16 = jnp.zeros((16,), jnp.float32)

    def zrow(i, _):
        for j in range(D // 16):
            rows[i, pl.ds(j * 16, 16)] = zero16
        return _

    lax.fori_loop(0, CH, zrow, 0)
    _over_row_chunks(s, lambda base: _copy_rows(
        lambda b, n: rows.at[pl.ds(0, n)],
        lambda b, n: acc.at[pl.ds(b, n)], base, BR, CH))
    plsc.subcore_barrier()

    cbase = wid * CPW

    def chunk(k):
        off = k * CH
        pltpu.sync_copy(src_hbm.at[pl.ds(off, CH)], idx_s)
        pltpu.sync_copy(dst_hbm.at[pl.ds(off, CH)], idx_d)
        pltpu.sync_copy(g_hbm.at[idx_s], rows)
        pltpu.sync_copy(rows, acc.at[idx_d], add=True)

    def body(k, _):
        chunk(cbase + k)
        return _

    lax.fori_loop(0, CPW, body, 0)

    @pl.when(wid < EXTRA)
    def _():
        chunk(NW * CPW + wid)

    plsc.subcore_barrier()
    _over_row_chunks(s, lambda base: _copy_rows(
        lambda b, n: acc.at[pl.ds(b, n)],
        lambda b, n: out_hbm.at[c, pl.ds(b, n)], base, BR, CH))


def _dis(deg_ref):
    return lax.rsqrt(deg_ref[:, 0:1] + deg_ref[:, 1:2] + 1.0)


def _first_tc(deg_ref, x_ref, w_ref, o_ref):
    o_ref[...] = jnp.dot(x_ref[...], w_ref[...],
                         preferred_element_type=jnp.float32) * _dis(deg_ref)


def _mid_tc(deg_ref, s_ref, g_ref, b_ref, w_ref, o_ref):
    dis = _dis(deg_ref)
    h = dis * (s_ref[0] + s_ref[1] + g_ref[...]) + b_ref[...]
    h = jnp.where(h >= 0.0, h, 0.01 * h)
    o_ref[...] = jnp.dot(h, w_ref[...],
                         preferred_element_type=jnp.float32) * dis


def _last_tc(deg_ref, s_ref, g_ref, b_ref, w_ref, bfc_ref, o_ref):
    dis = _dis(deg_ref)
    h = dis * (s_ref[0] + s_ref[1] + g_ref[...]) + b_ref[...]
    h = jnp.where(h >= 0.0, h, 0.01 * h)
    o_ref[...] = jnp.dot(h, w_ref[...],
                         preferred_element_type=jnp.float32) + bfc_ref[...]


_GRID = (N // BR,)
_deg_spec = pl.BlockSpec((BR, 2), lambda i: (i, 0))
_row_spec = pl.BlockSpec((BR, D), lambda i: (i, 0))
_s_spec = pl.BlockSpec((NC, BR, D), lambda i: (0, i, 0))
_w_spec = pl.BlockSpec((D, D), lambda i: (0, 0))
_b_spec = pl.BlockSpec((1, D), lambda i: (0, 0))
_out_shape = jax.ShapeDtypeStruct((N, D), jnp.float32)

_first_call = pl.pallas_call(
    _first_tc, grid=_GRID,
    in_specs=[_deg_spec, _row_spec, _w_spec],
    out_specs=_row_spec, out_shape=_out_shape)

_mid_call = pl.pallas_call(
    _mid_tc, grid=_GRID,
    in_specs=[_deg_spec, _s_spec, _row_spec, _b_spec, _w_spec],
    out_specs=_row_spec, out_shape=_out_shape)

_last_call = pl.pallas_call(
    _last_tc, grid=_GRID,
    in_specs=[_deg_spec, _s_spec, _row_spec, _b_spec, _w_spec, _b_spec],
    out_specs=_row_spec, out_shape=_out_shape)


def kernel(x, edge_index, W1, b1, W2, b2, Wfc, bfc):
    ei = edge_index.astype(jnp.int32)
    src, dst = ei[0], ei[1]
    onehot = jnp.repeat(jnp.eye(8, dtype=jnp.float32), 16, axis=1)
    degp = _deg_sc(dst, onehot)
    deg = degp.reshape(NC, PK, 8, 16)[:, :, :, 0].reshape(NC, NPAD)[:, :N]
    degT = jnp.transpose(deg)
    g1 = _first_call(degT, x, W1)
    s1 = _agg_sc(g1, src, dst)
    g2 = _mid_call(degT, s1, g1, b1.reshape(1, D), W2)
    s2 = _agg_sc(g2, src, dst)
    out = _last_call(degT, s2, g2, b2.reshape(1, D), Wfc, bfc.reshape(1, D))
    return out


# speedup vs baseline: 12.6301x; 2.6885x over previous
import functools

import jax
import jax.numpy as jnp
from jax import lax
from jax.experimental import pallas as pl
from jax.experimental.pallas import tpu as pltpu
from jax.experimental.pallas import tpu_sc as plsc

N = 10000
E = 320000
D = 128
NC, NS = 2, 16
NW = NC * NS
CH = 128
NCHUNKS = E // CH
CPW = NCHUNKS // NW
EXTRA = NCHUNKS - CPW * NW
BR = 400
NRC = N // BR

_mesh = plsc.VectorSubcoreMesh(core_axis_name="c", subcore_axis_name="s")


def _worker_id():
    return lax.axis_index("s") * NC + lax.axis_index("c")


def _copy_rows(src_at, dst_at, base, total, piece):
    off = 0
    while off < total:
        n = min(piece, total - off)
        pltpu.sync_copy(src_at(base + off, n), dst_at(base + off, n))
        off += n


def _over_row_chunks(s, fn):
    fn(s * BR)

    @pl.when(s < NRC - NS)
    def _():
        fn((s + NS) * BR)


NPAD = 10240
PK = NPAD // 8
PKS = PK // NS
TREP = 256


@functools.partial(
    pl.kernel,
    out_type=jax.ShapeDtypeStruct((NC, PK, 128), jnp.float32),
    mesh=_mesh,
    scratch_types=[
        pltpu.VMEM((CH,), jnp.int32),
        pltpu.VMEM((CH,), jnp.int32),
        pltpu.VMEM((CH,), jnp.int32),
        pltpu.VMEM((CH, D), jnp.float32),
        pltpu.VMEM_SHARED((PK, 128), jnp.float32),
    ],
)
def _deg_sc(dst_hbm, table_hbm, out_hbm, idx_v, idxhi, idxlo, rows, acc):
    c = lax.axis_index("c")
    s = lax.axis_index("s")
    wid = _worker_id()

    zero16 = jnp.zeros((16,), jnp.float32)

    def zrow(i, _):
        for j in range(D // 16):
            rows[i, pl.ds(j * 16, 16)] = zero16
        return _

    lax.fori_loop(0, CH, zrow, 0)
    pltpu.sync_copy(rows.at[pl.ds(0, PKS)], acc.at[pl.ds(s * PKS, PKS)])
    plsc.subcore_barrier()

    cbase = wid * CPW

    def chunk(k):
        off = k * CH
        pltpu.sync_copy(dst_hbm.at[pl.ds(off, CH)], idx_v)
        for j in range(CH // 16):
            v = idx_v[pl.ds(j * 16, 16)]
            idxhi[pl.ds(j * 16, 16)] = lax.shift_right_logical(v, 3)
            idxlo[pl.ds(j * 16, 16)] = lax.bitwise_and(v, TREP * 8 - 1)
        pltpu.sync_copy(table_hbm.at[idxlo], rows)
        pltpu.sync_copy(rows, acc.at[idxhi], add=True)

    def body(k, _):
        chunk(cbase + k)
        return _

    lax.fori_loop(0, CPW, body, 0)

    @pl.when(wid < EXTRA)
    def _():
        chunk(NW * CPW + wid)

    plsc.subcore_barrier()
    pltpu.sync_copy(acc.at[pl.ds(s * PKS, PKS)], out_hbm.at[c, pl.ds(s * PKS, PKS)])


@functools.partial(
    pl.kernel,
    out_type=jax.ShapeDtypeStruct((NC, N, D), jnp.float32),
    mesh=_mesh,
    scratch_types=[
        pltpu.VMEM((CH,), jnp.int32),
        pltpu.VMEM((CH,), jnp.int32),
        pltpu.VMEM((CH, D), jnp.float32),
        pltpu.VMEM_SHARED((N, D), jnp.float32),
    ],
)
def _agg_sc(g_hbm, src_hbm, dst_hbm, out_hbm, idx_s, idx_d, rows, acc):
    c = lax.axis_index("c")
    s = lax.axis_index("s")
    wid = _worker_id()

    zero16 = jnp.zeros((16,), jnp.float32)

    def zrow(i, _):
        for j in range(D // 16):
            rows[i, pl.ds(j * 16, 16)] = zero16
        return _

    lax.fori_loop(0, CH, zrow, 0)
    _over_row_chunks(s, lambda base: _copy_rows(
        lambda b, n: rows.at[pl.ds(0, n)],
        lambda b, n: acc.at[pl.ds(b, n)], base, BR, CH))
    plsc.subcore_barrier()

    cbase = wid * CPW

    def chunk(k):
        off = k * CH
        pltpu.sync_copy(src_hbm.at[pl.ds(off, CH)], idx_s)
        pltpu.sync_copy(dst_hbm.at[pl.ds(off, CH)], idx_d)
        pltpu.sync_copy(g_hbm.at[idx_s], rows)
        pltpu.sync_copy(rows, acc.at[idx_d], add=True)

    def body(k, _):
        chunk(cbase + k)
        return _

    lax.fori_loop(0, CPW, body, 0)

    @pl.when(wid < EXTRA)
    def _():
        chunk(NW * CPW + wid)

    plsc.subcore_barrier()
    _over_row_chunks(s, lambda base: _copy_rows(
        lambda b, n: acc.at[pl.ds(b, n)],
        lambda b, n: out_hbm.at[c, pl.ds(b, n)], base, BR, CH))


def _dis(deg_ref):
    return lax.rsqrt(deg_ref[:, 0:1] + deg_ref[:, 1:2] + 1.0)


def _first_tc(deg_ref, x_ref, w_ref, o_ref):
    o_ref[...] = jnp.dot(x_ref[...], w_ref[...],
                         preferred_element_type=jnp.float32) * _dis(deg_ref)


def _mid_tc(deg_ref, s_ref, g_ref, b_ref, w_ref, o_ref):
    dis = _dis(deg_ref)
    h = dis * (s_ref[0] + s_ref[1] + g_ref[...]) + b_ref[...]
    h = jnp.where(h >= 0.0, h, 0.01 * h)
    o_ref[...] = jnp.dot(h, w_ref[...],
                         preferred_element_type=jnp.float32) * dis


def _last_tc(deg_ref, s_ref, g_ref, b_ref, w_ref, bfc_ref, o_ref):
    dis = _dis(deg_ref)
    h = dis * (s_ref[0] + s_ref[1] + g_ref[...]) + b_ref[...]
    h = jnp.where(h >= 0.0, h, 0.01 * h)
    o_ref[...] = jnp.dot(h, w_ref[...],
                         preferred_element_type=jnp.float32) + bfc_ref[...]


_GRID = (N // BR,)
_deg_spec = pl.BlockSpec((BR, 2), lambda i: (i, 0))
_row_spec = pl.BlockSpec((BR, D), lambda i: (i, 0))
_s_spec = pl.BlockSpec((NC, BR, D), lambda i: (0, i, 0))
_w_spec = pl.BlockSpec((D, D), lambda i: (0, 0))
_b_spec = pl.BlockSpec((1, D), lambda i: (0, 0))
_out_shape = jax.ShapeDtypeStruct((N, D), jnp.float32)

_first_call = pl.pallas_call(
    _first_tc, grid=_GRID,
    in_specs=[_deg_spec, _row_spec, _w_spec],
    out_specs=_row_spec, out_shape=_out_shape)

_mid_call = pl.pallas_call(
    _mid_tc, grid=_GRID,
    in_specs=[_deg_spec, _s_spec, _row_spec, _b_spec, _w_spec],
    out_specs=_row_spec, out_shape=_out_shape)

_last_call = pl.pallas_call(
    _last_tc, grid=_GRID,
    in_specs=[_deg_spec, _s_spec, _row_spec, _b_spec, _w_spec, _b_spec],
    out_specs=_row_spec, out_shape=_out_shape)


def kernel(x, edge_index, W1, b1, W2, b2, Wfc, bfc):
    ei = edge_index.astype(jnp.int32)
    src, dst = ei[0], ei[1]
    onehot = jnp.tile(jnp.repeat(jnp.eye(8, dtype=jnp.float32), 16, axis=1),
                      (TREP, 1))
    degp = _deg_sc(dst, onehot)
    deg = degp.reshape(NC, PK, 8, 16)[:, :, :, 0].reshape(NC, NPAD)[:, :N]
    degT = jnp.transpose(deg)
    g1 = _first_call(degT, x, W1)
    s1 = _agg_sc(g1, src, dst)
    g2 = _mid_call(degT, s1, g1, b1.reshape(1, D), W2)
    s2 = _agg_sc(g2, src, dst)
    out = _last_call(degT, s2, g2, b2.reshape(1, D), Wfc, bfc.reshape(1, D))
    return out
